# baseline (speedup 1.0000x reference)
import functools

import jax
import jax.numpy as jnp
from jax import lax
from jax.experimental import pallas as pl
from jax.experimental.pallas import tpu as pltpu
from jax.experimental.pallas import tpu_sc as plsc

N, D, E, OUT = 2048, 1024, 8, 128
EPS = 2.220446049250313e-16
NEG_INF = float("-inf")

BLK = 256
NBLK = 24
SROWS = NBLK * BLK
NP = 2 * N
PB = 256
NW = 32
CH = N // NW


def _route_body(x_ref, wg_ref, p0_ref, p1_ref, g_ref, bexp_ref):
    x = x_ref[...]
    logits = jnp.dot(x, wg_ref[...], preferred_element_type=jnp.float32)
    iota_e = lax.broadcasted_iota(jnp.int32, (N, E), 1)
    m1 = jnp.max(logits, axis=1, keepdims=True)
    e0 = jnp.min(jnp.where(logits == m1, iota_e, E), axis=1, keepdims=True)
    masked = jnp.where(iota_e == e0, NEG_INF, logits)
    m2 = jnp.max(masked, axis=1, keepdims=True)
    e1 = jnp.min(jnp.where(masked == m2, iota_e, E), axis=1, keepdims=True)
    t = jnp.exp(m2 - m1)
    g_ref[...] = jnp.concatenate([1.0 / (1.0 + t), t / (1.0 + t)], axis=1)

    oh0 = iota_e == e0
    oh1 = iota_e == e1
    cnt = jnp.where(oh0, 1.0, 0.0) + jnp.where(oh1, 1.0, 0.0)
    cnt_bf = cnt.astype(jnp.bfloat16)
    r = lax.broadcasted_iota(jnp.int32, (PB, PB), 0)
    c = lax.broadcasted_iota(jnp.int32, (PB, PB), 1)
    tri = jnp.where(c < r, 1.0, 0.0).astype(jnp.bfloat16)
    parts = []
    off = jnp.zeros((1, E), jnp.float32)
    for b in range(N // PB):
        blk_bf = cnt_bf[b * PB:(b + 1) * PB, :]
        parts.append(jnp.dot(tri, blk_bf, preferred_element_type=jnp.float32) + off)
        off = off + jnp.sum(cnt[b * PB:(b + 1) * PB, :], axis=0, keepdims=True)
    prefix = jnp.concatenate(parts, axis=0)
    totals = off

    pb = jnp.ceil(totals * (1.0 / BLK))
    inc = pb
    for sh in (1, 2, 4):
        inc = inc + jnp.concatenate(
            [jnp.zeros((1, sh), jnp.float32), inc[:, :-sh]], axis=1)
    base_blk = inc - pb
    posval = prefix + base_blk * float(BLK)
    p0 = jnp.sum(jnp.where(oh0, posval, 0.0), axis=1, keepdims=True)
    p1 = jnp.sum(jnp.where(oh1, posval, 0.0), axis=1, keepdims=True)
    p0_ref[...] = p0.astype(jnp.int32)
    p1_ref[...] = p1.astype(jnp.int32)

    bio = lax.broadcasted_iota(jnp.int32, (NBLK, E), 0)
    eio = lax.broadcasted_iota(jnp.int32, (NBLK, E), 1)
    bb = base_blk.astype(jnp.int32)
    pbi = pb.astype(jnp.int32)
    cond = (bio >= bb) & (bio < bb + pbi)
    bexp_ref[...] = jnp.sum(jnp.where(cond, eio, 0), axis=1, keepdims=True)


def _route(x, w_gate):
    return pl.pallas_call(
        _route_body,
        out_shape=[
            jax.ShapeDtypeStruct((N, 1), jnp.int32),
            jax.ShapeDtypeStruct((N, 1), jnp.int32),
            jax.ShapeDtypeStruct((N, 2), jnp.float32),
            jax.ShapeDtypeStruct((NBLK, 1), jnp.int32),
        ],
    )(x, w_gate)


@functools.lru_cache(maxsize=1)
def _sc_dispatch():
    mesh = plsc.VectorSubcoreMesh(core_axis_name="c", subcore_axis_name="s")

    @functools.partial(
        pl.kernel,
        out_type=jax.ShapeDtypeStruct((SROWS, D), jnp.float32),
        mesh=mesh,
        scratch_types=[
            pltpu.VMEM((CH,), jnp.int32),
            pltpu.VMEM((CH,), jnp.int32),
            pltpu.VMEM((CH, D), jnp.float32),
            pltpu.SemaphoreType.DMA,
        ],
    )
    def sc_dispatch(x_hbm, p0_hbm, p1_hbm, xs_hbm, i0_v, i1_v, rows_v, sem):
        w = lax.axis_index("s") * 2 + lax.axis_index("c")
        base = w * CH
        pltpu.sync_copy(p0_hbm.at[pl.ds(base, CH)], i0_v)
        pltpu.sync_copy(p1_hbm.at[pl.ds(base, CH)], i1_v)
        pltpu.sync_copy(x_hbm.at[pl.ds(base, CH)], rows_v)
        c0 = pltpu.async_copy(rows_v, xs_hbm.at[i0_v], sem)
        c1 = pltpu.async_copy(rows_v, xs_hbm.at[i1_v], sem)
        c0.wait()
        c1.wait()

    return sc_dispatch


def _dispatch(x, p0, p1):
    return _sc_dispatch()(x, p0, p1)


def _gmm_body(be_ref, x_ref, w_ref, b_ref, y_ref):
    del be_ref
    w = w_ref[0].astype(jnp.bfloat16)
    xb = x_ref[...].astype(jnp.bfloat16)
    y_ref[...] = jnp.dot(xb, w, preferred_element_type=jnp.float32) + b_ref[0]


def _gmm(bexp, xs, W_exp, b_exp):
    grid_spec = pltpu.PrefetchScalarGridSpec(
        num_scalar_prefetch=1,
        grid=(NBLK,),
        in_specs=[
            pl.BlockSpec((BLK, D), lambda b, be: (b, 0)),
            pl.BlockSpec((1, D, OUT), lambda b, be: (be[b], 0, 0)),
            pl.BlockSpec((1, 1, OUT), lambda b, be: (be[b], 0, 0)),
        ],
        out_specs=pl.BlockSpec((BLK, OUT), lambda b, be: (b, 0)),
    )
    return pl.pallas_call(
        _gmm_body,
        grid_spec=grid_spec,
        out_shape=jax.ShapeDtypeStruct((SROWS, OUT), jnp.float32),
    )(bexp, xs, W_exp, b_exp.reshape(E, 1, OUT))


@functools.lru_cache(maxsize=1)
def _sc_collect():
    mesh = plsc.VectorSubcoreMesh(core_axis_name="c", subcore_axis_name="s")

    @functools.partial(
        pl.kernel,
        out_type=jax.ShapeDtypeStruct((NP, OUT), jnp.float32),
        mesh=mesh,
        scratch_types=[
            pltpu.VMEM((CH,), jnp.int32),
            pltpu.VMEM((CH,), jnp.int32),
            pltpu.VMEM((CH, OUT), jnp.float32),
            pltpu.VMEM((CH, OUT), jnp.float32),
            pltpu.SemaphoreType.DMA,
        ],
    )
    def sc_collect(ys_hbm, p0_hbm, p1_hbm, yg_hbm, i0_v, i1_v, r0_v, r1_v, sem):
        w = lax.axis_index("s") * 2 + lax.axis_index("c")
        base = w * CH
        pltpu.sync_copy(p0_hbm.at[pl.ds(base, CH)], i0_v)
        pltpu.sync_copy(p1_hbm.at[pl.ds(base, CH)], i1_v)
        c0 = pltpu.async_copy(ys_hbm.at[i0_v], r0_v, sem)
        c1 = pltpu.async_copy(ys_hbm.at[i1_v], r1_v, sem)
        c0.wait()
        c1.wait()
        pltpu.sync_copy(r0_v, yg_hbm.at[pl.ds(base, CH)])
        pltpu.sync_copy(r1_v, yg_hbm.at[pl.ds(N + base, CH)])

    return sc_collect


def _collect(ys, p0, p1):
    return _sc_collect()(ys, p0, p1)


CB = 256


def _combine_body(y0_ref, y1_ref, g_ref, o_ref):
    g0 = g_ref[:, 0:1]
    g1 = g_ref[:, 1:2]
    acc = g0 * jnp.exp(y0_ref[...]) + g1 * jnp.exp(y1_ref[...])
    acc = jnp.where(acc == 0.0, EPS, acc)
    o_ref[...] = jnp.log(acc)


def _combine(yg, g2):
    nb = N // CB
    return pl.pallas_call(
        _combine_body,
        grid=(nb,),
        in_specs=[
            pl.BlockSpec((CB, OUT), lambda i: (i, 0)),
            pl.BlockSpec((CB, OUT), lambda i: (i + nb, 0)),
            pl.BlockSpec((CB, 2), lambda i: (i, 0)),
        ],
        out_specs=pl.BlockSpec((CB, OUT), lambda i: (i, 0)),
        out_shape=jax.ShapeDtypeStruct((N, OUT), jnp.float32),
    )(yg, yg, g2)


def kernel(x, w_gate, W_exp, b_exp):
    p0, p1, g2, bexp = _route(x, w_gate)
    p0 = p0.reshape(N)
    p1 = p1.reshape(N)
    xs = _dispatch(x, p0, p1)
    ys = _gmm(bexp.reshape(NBLK), xs, W_exp, b_exp)
    yg = _collect(ys, p0, p1)
    return _combine(yg, g2)


# speedup vs baseline: 4.9259x; 4.9259x over previous
import jax
import jax.numpy as jnp
from jax import lax
from jax.experimental import pallas as pl
from jax.experimental.pallas import tpu as pltpu

N, D, E, OUT = 2048, 1024, 8, 128
EPS = 2.220446049250313e-16
TBLK = 256
NEG_INF = float("-inf")


def _moe_dense_body(x_ref, wg_ref, w_ref, b_ref, o_ref, wbf_ref):
    @pl.when(pl.program_id(0) == 0)
    def _():
        for e in range(E):
            wbf_ref[:, e * OUT:(e + 1) * OUT] = w_ref[e].astype(jnp.bfloat16)

    x = x_ref[...]
    xb = x.astype(jnp.bfloat16)
    logits = jnp.dot(xb, wg_ref[...].astype(jnp.bfloat16),
                     preferred_element_type=jnp.float32)
    iota_e = lax.broadcasted_iota(jnp.int32, (TBLK, E), 1)
    m1 = jnp.max(logits, axis=1, keepdims=True)
    e0 = jnp.min(jnp.where(logits == m1, iota_e, E), axis=1, keepdims=True)
    masked = jnp.where(iota_e == e0, NEG_INF, logits)
    m2 = jnp.max(masked, axis=1, keepdims=True)
    e1 = jnp.min(jnp.where(masked == m2, iota_e, E), axis=1, keepdims=True)
    t = jnp.exp(m2 - m1)
    g0 = 1.0 / (1.0 + t)
    g1 = t / (1.0 + t)
    big = jnp.dot(xb, wbf_ref[...], preferred_element_type=jnp.float32)
    acc = jnp.zeros((TBLK, OUT), jnp.float32)
    for e in range(E):
        o = big[:, e * OUT:(e + 1) * OUT] + b_ref[e:e + 1, :]
        ge = jnp.where(e0 == e, g0, jnp.where(e1 == e, g1, 0.0))
        acc = acc + ge * jnp.exp(o)
    acc = jnp.where(acc == 0.0, EPS, acc)
    o_ref[...] = jnp.log(acc)


def kernel(x, w_gate, W_exp, b_exp):
    return pl.pallas_call(
        _moe_dense_body,
        grid=(N // TBLK,),
        in_specs=[
            pl.BlockSpec((TBLK, D), lambda i: (i, 0)),
            pl.BlockSpec((D, E), lambda i: (0, 0)),
            pl.BlockSpec((E, D, OUT), lambda i: (0, 0, 0)),
            pl.BlockSpec((E, OUT), lambda i: (0, 0)),
        ],
        out_specs=pl.BlockSpec((TBLK, OUT), lambda i: (i, 0)),
        out_shape=jax.ShapeDtypeStruct((N, OUT), jnp.float32),
        scratch_shapes=[pltpu.VMEM((D, E * OUT), jnp.bfloat16)],
    )(x, w_gate, W_exp, b_exp)


# speedup vs baseline: 5.3072x; 1.0774x over previous
import jax
import jax.numpy as jnp
from jax import lax
from jax.experimental import pallas as pl
from jax.experimental.pallas import tpu as pltpu

N, D, E, OUT = 2048, 1024, 8, 128
EPS = 2.220446049250313e-16
TBLK = 512
NEG_INF = float("-inf")


def _moe_dense_body(x_ref, wg_ref, w_ref, b_ref, o_ref, wbf_ref):
    @pl.when(pl.program_id(0) == 0)
    def _():
        for e in range(E):
            wbf_ref[:, e * OUT:(e + 1) * OUT] = w_ref[e].astype(jnp.bfloat16)

    x = x_ref[...]
    xb = x.astype(jnp.bfloat16)
    logits = jnp.dot(xb, wg_ref[...].astype(jnp.bfloat16),
                     preferred_element_type=jnp.float32)
    iota_e = lax.broadcasted_iota(jnp.int32, (TBLK, E), 1)
    m1 = jnp.max(logits, axis=1, keepdims=True)
    e0 = jnp.min(jnp.where(logits == m1, iota_e, E), axis=1, keepdims=True)
    masked = jnp.where(iota_e == e0, NEG_INF, logits)
    m2 = jnp.max(masked, axis=1, keepdims=True)
    e1 = jnp.min(jnp.where(masked == m2, iota_e, E), axis=1, keepdims=True)
    t = jnp.exp(m2 - m1)
    g0 = 1.0 / (1.0 + t)
    g1 = t / (1.0 + t)
    big = jnp.dot(xb, wbf_ref[...], preferred_element_type=jnp.float32)
    acc = jnp.zeros((TBLK, OUT), jnp.float32)
    for e in range(E):
        o = big[:, e * OUT:(e + 1) * OUT] + b_ref[e:e + 1, :]
        ge = jnp.where(e0 == e, g0, jnp.where(e1 == e, g1, 0.0))
        acc = acc + ge * jnp.exp(o)
    acc = jnp.where(acc == 0.0, EPS, acc)
    o_ref[...] = jnp.log(acc)


def kernel(x, w_gate, W_exp, b_exp):
    return pl.pallas_call(
        _moe_dense_body,
        grid=(N // TBLK,),
        in_specs=[
            pl.BlockSpec((TBLK, D), lambda i: (i, 0)),
            pl.BlockSpec((D, E), lambda i: (0, 0)),
            pl.BlockSpec((E, D, OUT), lambda i: (0, 0, 0)),
            pl.BlockSpec((E, OUT), lambda i: (0, 0)),
        ],
        out_specs=pl.BlockSpec((TBLK, OUT), lambda i: (i, 0)),
        out_shape=jax.ShapeDtypeStruct((N, OUT), jnp.float32),
        scratch_shapes=[pltpu.VMEM((D, E * OUT), jnp.bfloat16)],
    )(x, w_gate, W_exp, b_exp)


# speedup vs baseline: 5.4391x; 1.0249x over previous
import jax
import jax.numpy as jnp
from jax import lax
from jax.experimental import pallas as pl
from jax.experimental.pallas import tpu as pltpu

N, D, E, OUT = 2048, 1024, 8, 128
EPS = 2.220446049250313e-16
TBLK = 1024
NEG_INF = float("-inf")


def _moe_dense_body(x_ref, wg_ref, w_ref, b_ref, o_ref, wbf_ref):
    @pl.when(pl.program_id(0) == 0)
    def _():
        for e in range(E):
            wbf_ref[:, e * OUT:(e + 1) * OUT] = w_ref[e].astype(jnp.bfloat16)

    x = x_ref[...]
    xb = x.astype(jnp.bfloat16)
    logits = jnp.dot(xb, wg_ref[...].astype(jnp.bfloat16),
                     preferred_element_type=jnp.float32)
    iota_e = lax.broadcasted_iota(jnp.int32, (TBLK, E), 1)
    m1 = jnp.max(logits, axis=1, keepdims=True)
    e0 = jnp.min(jnp.where(logits == m1, iota_e, E), axis=1, keepdims=True)
    masked = jnp.where(iota_e == e0, NEG_INF, logits)
    m2 = jnp.max(masked, axis=1, keepdims=True)
    e1 = jnp.min(jnp.where(masked == m2, iota_e, E), axis=1, keepdims=True)
    t = jnp.exp(m2 - m1)
    g0 = 1.0 / (1.0 + t)
    g1 = t / (1.0 + t)
    big = jnp.dot(xb, wbf_ref[...], preferred_element_type=jnp.float32)
    acc = jnp.zeros((TBLK, OUT), jnp.float32)
    for e in range(E):
        o = big[:, e * OUT:(e + 1) * OUT] + b_ref[e:e + 1, :]
        ge = jnp.where(e0 == e, g0, jnp.where(e1 == e, g1, 0.0))
        acc = acc + ge * jnp.exp(o)
    acc = jnp.where(acc == 0.0, EPS, acc)
    o_ref[...] = jnp.log(acc)


def kernel(x, w_gate, W_exp, b_exp):
    return pl.pallas_call(
        _moe_dense_body,
        grid=(N // TBLK,),
        in_specs=[
            pl.BlockSpec((TBLK, D), lambda i: (i, 0)),
            pl.BlockSpec((D, E), lambda i: (0, 0)),
            pl.BlockSpec((E, D, OUT), lambda i: (0, 0, 0)),
            pl.BlockSpec((E, OUT), lambda i: (0, 0)),
        ],
        out_specs=pl.BlockSpec((TBLK, OUT), lambda i: (i, 0)),
        out_shape=jax.ShapeDtypeStruct((N, OUT), jnp.float32),
        scratch_shapes=[pltpu.VMEM((D, E * OUT), jnp.bfloat16)],
    )(x, w_gate, W_exp, b_exp)
